# Initial kernel scaffold; baseline (speedup 1.0000x reference)
import jax
import jax.numpy as jnp
from jax.experimental import pallas as pl


def kernel(x, gate_w, gate_b):
    raise NotImplementedError("write your pallas kernel here")


# speedup vs baseline: 1.7279x; 1.7279x over previous
import functools

import jax
import jax.numpy as jnp
from jax.experimental import pallas as pl
from jax.experimental.pallas import tpu as pltpu

_BATCH, _SEQ, _D = 4, 4096, 2048
_E = 64
_TOKENS = _BATCH * _SEQ
_TB = 1024


def _router_kernel(x_ref, wt_ref, b_ref, w_out_ref, i_out_ref, loss_ref,
                   acc_ref, *, n_steps, n_tokens):
    step = pl.program_id(0)

    logits = jnp.dot(x_ref[...], wt_ref[...],
                     preferred_element_type=jnp.float32) + b_ref[...]

    tb = logits.shape[0]
    iota = jax.lax.broadcasted_iota(jnp.int32, (tb, _E), 1)

    m1 = jnp.max(logits, axis=-1, keepdims=True)
    i1 = jnp.min(jnp.where(logits == m1, iota, _E), axis=-1, keepdims=True)
    masked = jnp.where(iota == i1, -jnp.inf, logits)
    m2 = jnp.max(masked, axis=-1, keepdims=True)
    i2 = jnp.min(jnp.where(masked == m2, iota, _E), axis=-1, keepdims=True)

    e = jnp.exp(m2 - m1)
    denom = 1.0 + e
    w_out_ref[...] = jnp.concatenate([1.0 / denom, e / denom], axis=1)
    i_out_ref[...] = jnp.concatenate([i1, i2], axis=1)

    probs = jnp.exp(logits - m1)
    probs = probs / jnp.sum(probs, axis=-1, keepdims=True)
    part = jnp.sum(probs, axis=0, keepdims=True)

    @pl.when(step == 0)
    def _():
        acc_ref[...] = jnp.zeros_like(acc_ref)

    acc_ref[...] += part

    @pl.when(step == n_steps - 1)
    def _():
        usage = acc_ref[...] * (1.0 / n_tokens)
        ssq = jnp.sum(usage * usage, axis=1, keepdims=True)
        loss_ref[...] = _E * ssq - 1.0


def kernel(x, gate_w, gate_b):
    xf = x.reshape(_TOKENS, _D)
    wt = gate_w.T
    b2 = gate_b.reshape(1, _E)
    n_steps = _TOKENS // _TB

    weights, indices, loss = pl.pallas_call(
        functools.partial(_router_kernel, n_steps=n_steps, n_tokens=_TOKENS),
        grid=(n_steps,),
        in_specs=[
            pl.BlockSpec((_TB, _D), lambda i: (i, 0)),
            pl.BlockSpec((_D, _E), lambda i: (0, 0)),
            pl.BlockSpec((1, _E), lambda i: (0, 0)),
        ],
        out_specs=[
            pl.BlockSpec((_TB, 2), lambda i: (i, 0)),
            pl.BlockSpec((_TB, 2), lambda i: (i, 0)),
            pl.BlockSpec((1, 1), lambda i: (0, 0)),
        ],
        out_shape=[
            jax.ShapeDtypeStruct((_TOKENS, 2), jnp.float32),
            jax.ShapeDtypeStruct((_TOKENS, 2), jnp.int32),
            jax.ShapeDtypeStruct((1, 1), jnp.float32),
        ],
        scratch_shapes=[pltpu.VMEM((1, _E), jnp.float32)],
    )(xf, wt, b2)

    return (weights.reshape(_BATCH, _SEQ, 2),
            indices.reshape(_BATCH, _SEQ, 2),
            loss[0, 0])


# speedup vs baseline: 1.8006x; 1.0421x over previous
import functools

import jax
import jax.numpy as jnp
from jax.experimental import pallas as pl
from jax.experimental.pallas import tpu as pltpu

_BATCH, _SEQ, _D = 4, 4096, 2048
_E = 64
_TOKENS = _BATCH * _SEQ
_TB = 2048


def _router_kernel(x_ref, wt_ref, b_ref, w_out_ref, i_out_ref, loss_ref,
                   acc_ref, *, n_steps, n_tokens):
    step = pl.program_id(0)

    logits = jnp.dot(x_ref[...], wt_ref[...],
                     preferred_element_type=jnp.float32) + b_ref[...]

    tb = logits.shape[0]
    iota = jax.lax.broadcasted_iota(jnp.int32, (tb, _E), 1)

    m1 = jnp.max(logits, axis=-1, keepdims=True)
    i1 = jnp.min(jnp.where(logits == m1, iota, _E), axis=-1, keepdims=True)
    masked = jnp.where(iota == i1, -jnp.inf, logits)
    m2 = jnp.max(masked, axis=-1, keepdims=True)
    i2 = jnp.min(jnp.where(masked == m2, iota, _E), axis=-1, keepdims=True)

    e = jnp.exp(m2 - m1)
    denom = 1.0 + e
    w_out_ref[...] = jnp.concatenate([1.0 / denom, e / denom], axis=1)
    i_out_ref[...] = jnp.concatenate([i1, i2], axis=1)

    probs = jnp.exp(logits - m1)
    probs = probs / jnp.sum(probs, axis=-1, keepdims=True)
    part = jnp.sum(probs, axis=0, keepdims=True)

    @pl.when(step == 0)
    def _():
        acc_ref[...] = jnp.zeros_like(acc_ref)

    acc_ref[...] += part

    @pl.when(step == n_steps - 1)
    def _():
        usage = acc_ref[...] * (1.0 / n_tokens)
        ssq = jnp.sum(usage * usage, axis=1, keepdims=True)
        loss_ref[...] = _E * ssq - 1.0


def kernel(x, gate_w, gate_b):
    xf = x.reshape(_TOKENS, _D)
    wt = gate_w.T
    b2 = gate_b.reshape(1, _E)
    n_steps = _TOKENS // _TB

    weights, indices, loss = pl.pallas_call(
        functools.partial(_router_kernel, n_steps=n_steps, n_tokens=_TOKENS),
        grid=(n_steps,),
        in_specs=[
            pl.BlockSpec((_TB, _D), lambda i: (i, 0)),
            pl.BlockSpec((_D, _E), lambda i: (0, 0)),
            pl.BlockSpec((1, _E), lambda i: (0, 0)),
        ],
        out_specs=[
            pl.BlockSpec((_TB, 2), lambda i: (i, 0)),
            pl.BlockSpec((_TB, 2), lambda i: (i, 0)),
            pl.BlockSpec((1, 1), lambda i: (0, 0)),
        ],
        out_shape=[
            jax.ShapeDtypeStruct((_TOKENS, 2), jnp.float32),
            jax.ShapeDtypeStruct((_TOKENS, 2), jnp.int32),
            jax.ShapeDtypeStruct((1, 1), jnp.float32),
        ],
        scratch_shapes=[pltpu.VMEM((1, _E), jnp.float32)],
    )(xf, wt, b2)

    return (weights.reshape(_BATCH, _SEQ, 2),
            indices.reshape(_BATCH, _SEQ, 2),
            loss[0, 0])
